# baseline (speedup 1.0000x reference)
import jax
import jax.numpy as jnp
from jax.experimental import pallas as pl
from jax.experimental.pallas import tpu as pltpu
from jax.experimental.pallas import tpu_sc as plsc

B, C, H, W = 16, 3, 224, 224
P = 14
K = 8192
D = 64
GH, GW = H // P, W // P
T = GH * GW
PATCH_DIM = C * P * P
BT = B * T

BLK = 256
NBLK = BT // BLK
KBLK = K // NBLK
GATHER_WIN = 128
GATHER_DIM = 128

_HBM = pl.BlockSpec(memory_space=pltpu.MemorySpace.HBM)


def _patchify_body(in_hbm, out_hbm, sem):
    for i in range(C * P):
        c, p1 = divmod(i, P)
        pltpu.make_async_copy(in_hbm.at[:, c, :, p1, :, :],
                              out_hbm.at[:, :, :, c, p1, :], sem).start()
    for i in range(C * P):
        c, p1 = divmod(i, P)
        pltpu.make_async_copy(in_hbm.at[:, c, :, p1, :, :],
                              out_hbm.at[:, :, :, c, p1, :], sem).wait()


def _unpatchify_body(in_hbm, out_hbm, sem):
    for i in range(C * P):
        c, p1 = divmod(i, P)
        pltpu.make_async_copy(in_hbm.at[:, :, :, c, p1, :],
                              out_hbm.at[:, c, :, p1, :, :], sem).start()
    for i in range(C * P):
        c, p1 = divmod(i, P)
        pltpu.make_async_copy(in_hbm.at[:, :, :, c, p1, :],
                              out_hbm.at[:, c, :, p1, :, :], sem).wait()


def _encode_body(p_ref, we_ref, be_ref, cb_ref, enc_ref, cbp_ref):
    enc_ref[...] = jnp.dot(p_ref[...], we_ref[...],
                           preferred_element_type=jnp.float32) + be_ref[...]
    cb = cb_ref[...]
    cbp_ref[...] = jnp.concatenate([cb, jnp.zeros_like(cb)], axis=1)


def _argmin_body(enc_ref, cbt_ref, rn_ref, cbn_ref, idx_ref):
    d2 = (rn_ref[...]
          - 2.0 * jnp.dot(enc_ref[...], cbt_ref[...],
                          preferred_element_type=jnp.float32)
          + cbn_ref[...])
    m = jnp.min(d2, axis=1, keepdims=True)
    iota = jax.lax.broadcasted_iota(jnp.int32, d2.shape, 1).astype(jnp.float32)
    idx = jnp.min(jnp.where(d2 == m, iota, jnp.float32(K)), axis=1,
                  keepdims=True)
    idx_ref[...] = idx.astype(jnp.int32)


def _decode_loss_body(enc_ref, embp_ref, p_ref, wd_ref, bd_ref,
                      emb_ref, dec_ref, sse_ref, sae_ref):
    emb = embp_ref[:, :D]
    emb_ref[...] = emb
    dec = jnp.dot(emb, wd_ref[...],
                  preferred_element_type=jnp.float32) + bd_ref[...]
    dec_ref[...] = dec

    @pl.when(pl.program_id(0) == 0)
    def _():
        sse_ref[...] = jnp.zeros_like(sse_ref)
        sae_ref[...] = jnp.zeros_like(sae_ref)

    diff = enc_ref[...] - emb
    sse_ref[...] += jnp.sum(diff * diff).reshape(1, 1)
    sae_ref[...] += jnp.sum(jnp.abs(dec - p_ref[...])).reshape(1, 1)


def _sc_gather(cb_pad, idx_row):
    mesh = plsc.VectorSubcoreMesh(core_axis_name="core",
                                  subcore_axis_name="subcore")

    @pl.kernel(out_type=jax.ShapeDtypeStruct((BT, GATHER_DIM), jnp.float32),
               mesh=mesh)
    def k(cb_hbm, i_hbm, o_hbm):
        def body(i_vmem, o_vmem):
            pltpu.sync_copy(cb_hbm.at[i_vmem.at[0]], o_vmem)

        pltpu.emit_pipeline(
            body,
            grid=(BT // GATHER_WIN,),
            in_specs=[pl.BlockSpec((1, GATHER_WIN), index_map=lambda i: (0, i))],
            out_specs=[pl.BlockSpec((GATHER_WIN, GATHER_DIM),
                                    index_map=lambda i: (i, 0))],
            core_axis_name=("core", "subcore"),
            dimension_semantics=(pltpu.PARALLEL,),
        )(i_hbm, o_hbm)

    return k(cb_pad, idx_row)


def kernel(inputs, W_enc, b_enc, codebook, W_dec, b_dec, commitment):
    inputs6 = inputs.reshape(B, C, GH, P, GW, P)

    patches6 = pl.pallas_call(
        _patchify_body,
        in_specs=[_HBM],
        out_specs=_HBM,
        out_shape=jax.ShapeDtypeStruct((B, GH, GW, C, P, P), jnp.float32),
        scratch_shapes=[pltpu.SemaphoreType.DMA],
    )(inputs6)
    patches = patches6.reshape(BT, PATCH_DIM)

    enc_flat, cb_pad = pl.pallas_call(
        _encode_body,
        grid=(NBLK,),
        in_specs=[
            pl.BlockSpec((BLK, PATCH_DIM), lambda i: (i, 0)),
            pl.BlockSpec((PATCH_DIM, D), lambda i: (0, 0)),
            pl.BlockSpec((1, D), lambda i: (0, 0)),
            pl.BlockSpec((KBLK, D), lambda i: (i, 0)),
        ],
        out_specs=[
            pl.BlockSpec((BLK, D), lambda i: (i, 0)),
            pl.BlockSpec((KBLK, GATHER_DIM), lambda i: (i, 0)),
        ],
        out_shape=[
            jax.ShapeDtypeStruct((BT, D), jnp.float32),
            jax.ShapeDtypeStruct((K, GATHER_DIM), jnp.float32),
        ],
    )(patches, W_enc, b_enc.reshape(1, D), codebook)

    rn = jnp.sum(enc_flat * enc_flat, axis=1, keepdims=True)
    cbn = jnp.sum(codebook * codebook, axis=1)[None, :]

    idx_col = pl.pallas_call(
        _argmin_body,
        grid=(NBLK,),
        in_specs=[
            pl.BlockSpec((BLK, D), lambda i: (i, 0)),
            pl.BlockSpec((D, K), lambda i: (0, 0)),
            pl.BlockSpec((BLK, 1), lambda i: (i, 0)),
            pl.BlockSpec((1, K), lambda i: (0, 0)),
        ],
        out_specs=pl.BlockSpec((BLK, 1), lambda i: (i, 0)),
        out_shape=jax.ShapeDtypeStruct((BT, 1), jnp.int32),
    )(enc_flat, codebook.T, rn, cbn)

    idxs = idx_col.reshape(B, T)

    emb_pad = _sc_gather(cb_pad, idx_col.reshape(1, BT))

    emb_flat, dec_flat, sse, sae = pl.pallas_call(
        _decode_loss_body,
        grid=(NBLK,),
        in_specs=[
            pl.BlockSpec((BLK, D), lambda i: (i, 0)),
            pl.BlockSpec((BLK, GATHER_DIM), lambda i: (i, 0)),
            pl.BlockSpec((BLK, PATCH_DIM), lambda i: (i, 0)),
            pl.BlockSpec((D, PATCH_DIM), lambda i: (0, 0)),
            pl.BlockSpec((1, PATCH_DIM), lambda i: (0, 0)),
        ],
        out_specs=[
            pl.BlockSpec((BLK, D), lambda i: (i, 0)),
            pl.BlockSpec((BLK, PATCH_DIM), lambda i: (i, 0)),
            pl.BlockSpec((1, 1), lambda i: (0, 0)),
            pl.BlockSpec((1, 1), lambda i: (0, 0)),
        ],
        out_shape=[
            jax.ShapeDtypeStruct((BT, D), jnp.float32),
            jax.ShapeDtypeStruct((BT, PATCH_DIM), jnp.float32),
            jax.ShapeDtypeStruct((1, 1), jnp.float32),
            jax.ShapeDtypeStruct((1, 1), jnp.float32),
        ],
    )(enc_flat, emb_pad, patches, W_dec, b_dec.reshape(1, PATCH_DIM))

    recon6 = pl.pallas_call(
        _unpatchify_body,
        in_specs=[_HBM],
        out_specs=_HBM,
        out_shape=jax.ShapeDtypeStruct((B, C, GH, P, GW, P), jnp.float32),
        scratch_shapes=[pltpu.SemaphoreType.DMA],
    )(dec_flat.reshape(B, GH, GW, C, P, P))

    recon = recon6.reshape(B, C, H, W)
    total_vq_loss = sse[0, 0] / (BT * D) * (1.0 + commitment)
    recon_loss = sae[0, 0] / (B * C * H * W)
    overall = total_vq_loss + recon_loss
    embedded_pt = emb_flat.reshape(B, T, D)
    return (overall, total_vq_loss, recon_loss, recon, embedded_pt, idxs)


# speedup vs baseline: 17.6458x; 17.6458x over previous
import jax
import jax.numpy as jnp
from jax.experimental import pallas as pl
from jax.experimental.pallas import tpu as pltpu
from jax.experimental.pallas import tpu_sc as plsc

B, C, H, W = 16, 3, 224, 224
P = 14
K = 8192
D = 64
GH, GW = H // P, W // P
T = GH * GW
PATCH_DIM = C * P * P
BT = B * T

BLK = 256
NBLK = BT // BLK
KBLK = K // NBLK
GATHER_WIN = 128
GATHER_DIM = 128

_HBM = pl.BlockSpec(memory_space=pltpu.MemorySpace.HBM)


def _patchify_body(in_ref, out_ref):
    x = in_ref[0]
    out_ref[0] = x.transpose(1, 3, 0, 2, 4).reshape(T, PATCH_DIM)


def _unpatchify_body(in_ref, out_ref):
    x = in_ref[0].reshape(GH, GW, C, P, P)
    out_ref[0] = x.transpose(2, 0, 3, 1, 4)


def _encode_body(p_ref, we_ref, be_ref, cb_ref, enc_ref, cbp_ref):
    enc_ref[...] = jnp.dot(p_ref[...], we_ref[...],
                           preferred_element_type=jnp.float32) + be_ref[...]
    cb = cb_ref[...]
    cbp_ref[...] = jnp.concatenate([cb, jnp.zeros_like(cb)], axis=1)


def _argmin_body(enc_ref, cbt_ref, rn_ref, cbn_ref, idx_ref):
    d2 = (rn_ref[...]
          - 2.0 * jnp.dot(enc_ref[...], cbt_ref[...],
                          preferred_element_type=jnp.float32)
          + cbn_ref[...])
    m = jnp.min(d2, axis=1, keepdims=True)
    iota = jax.lax.broadcasted_iota(jnp.int32, d2.shape, 1).astype(jnp.float32)
    idx = jnp.min(jnp.where(d2 == m, iota, jnp.float32(K)), axis=1,
                  keepdims=True)
    idx_ref[...] = idx.astype(jnp.int32)


def _decode_loss_body(enc_ref, embp_ref, p_ref, wd_ref, bd_ref,
                      emb_ref, dec_ref, sse_ref, sae_ref):
    emb = embp_ref[:, :D]
    emb_ref[...] = emb
    dec = jnp.dot(emb, wd_ref[...],
                  preferred_element_type=jnp.float32) + bd_ref[...]
    dec_ref[...] = dec

    @pl.when(pl.program_id(0) == 0)
    def _():
        sse_ref[...] = jnp.zeros_like(sse_ref)
        sae_ref[...] = jnp.zeros_like(sae_ref)

    diff = enc_ref[...] - emb
    sse_ref[...] += jnp.sum(diff * diff).reshape(1, 1)
    sae_ref[...] += jnp.sum(jnp.abs(dec - p_ref[...])).reshape(1, 1)


def _sc_gather(cb_pad, idx_row):
    mesh = plsc.VectorSubcoreMesh(core_axis_name="core",
                                  subcore_axis_name="subcore")

    @pl.kernel(out_type=jax.ShapeDtypeStruct((BT, GATHER_DIM), jnp.float32),
               mesh=mesh)
    def k(cb_hbm, i_hbm, o_hbm):
        def body(i_vmem, o_vmem):
            pltpu.sync_copy(cb_hbm.at[i_vmem.at[0]], o_vmem)

        pltpu.emit_pipeline(
            body,
            grid=(BT // GATHER_WIN,),
            in_specs=[pl.BlockSpec((1, GATHER_WIN), index_map=lambda i: (0, i))],
            out_specs=[pl.BlockSpec((GATHER_WIN, GATHER_DIM),
                                    index_map=lambda i: (i, 0))],
            core_axis_name=("core", "subcore"),
            dimension_semantics=(pltpu.PARALLEL,),
        )(i_hbm, o_hbm)

    return k(cb_pad, idx_row)


def kernel(inputs, W_enc, b_enc, codebook, W_dec, b_dec, commitment):
    inputs6 = inputs.reshape(B, C, GH, P, GW, P)

    patches3 = pl.pallas_call(
        _patchify_body,
        grid=(B,),
        in_specs=[pl.BlockSpec((1, C, GH, P, GW, P),
                               lambda i: (i, 0, 0, 0, 0, 0))],
        out_specs=pl.BlockSpec((1, T, PATCH_DIM), lambda i: (i, 0, 0)),
        out_shape=jax.ShapeDtypeStruct((B, T, PATCH_DIM), jnp.float32),
    )(inputs6)
    patches = patches3.reshape(BT, PATCH_DIM)

    enc_flat, cb_pad = pl.pallas_call(
        _encode_body,
        grid=(NBLK,),
        in_specs=[
            pl.BlockSpec((BLK, PATCH_DIM), lambda i: (i, 0)),
            pl.BlockSpec((PATCH_DIM, D), lambda i: (0, 0)),
            pl.BlockSpec((1, D), lambda i: (0, 0)),
            pl.BlockSpec((KBLK, D), lambda i: (i, 0)),
        ],
        out_specs=[
            pl.BlockSpec((BLK, D), lambda i: (i, 0)),
            pl.BlockSpec((KBLK, GATHER_DIM), lambda i: (i, 0)),
        ],
        out_shape=[
            jax.ShapeDtypeStruct((BT, D), jnp.float32),
            jax.ShapeDtypeStruct((K, GATHER_DIM), jnp.float32),
        ],
    )(patches, W_enc, b_enc.reshape(1, D), codebook)

    rn = jnp.sum(enc_flat * enc_flat, axis=1, keepdims=True)
    cbn = jnp.sum(codebook * codebook, axis=1)[None, :]

    idx_col = pl.pallas_call(
        _argmin_body,
        grid=(NBLK,),
        in_specs=[
            pl.BlockSpec((BLK, D), lambda i: (i, 0)),
            pl.BlockSpec((D, K), lambda i: (0, 0)),
            pl.BlockSpec((BLK, 1), lambda i: (i, 0)),
            pl.BlockSpec((1, K), lambda i: (0, 0)),
        ],
        out_specs=pl.BlockSpec((BLK, 1), lambda i: (i, 0)),
        out_shape=jax.ShapeDtypeStruct((BT, 1), jnp.int32),
    )(enc_flat, codebook.T, rn, cbn)

    idxs = idx_col.reshape(B, T)

    emb_pad = _sc_gather(cb_pad, idx_col.reshape(1, BT))

    emb_flat, dec_flat, sse, sae = pl.pallas_call(
        _decode_loss_body,
        grid=(NBLK,),
        in_specs=[
            pl.BlockSpec((BLK, D), lambda i: (i, 0)),
            pl.BlockSpec((BLK, GATHER_DIM), lambda i: (i, 0)),
            pl.BlockSpec((BLK, PATCH_DIM), lambda i: (i, 0)),
            pl.BlockSpec((D, PATCH_DIM), lambda i: (0, 0)),
            pl.BlockSpec((1, PATCH_DIM), lambda i: (0, 0)),
        ],
        out_specs=[
            pl.BlockSpec((BLK, D), lambda i: (i, 0)),
            pl.BlockSpec((BLK, PATCH_DIM), lambda i: (i, 0)),
            pl.BlockSpec((1, 1), lambda i: (0, 0)),
            pl.BlockSpec((1, 1), lambda i: (0, 0)),
        ],
        out_shape=[
            jax.ShapeDtypeStruct((BT, D), jnp.float32),
            jax.ShapeDtypeStruct((BT, PATCH_DIM), jnp.float32),
            jax.ShapeDtypeStruct((1, 1), jnp.float32),
            jax.ShapeDtypeStruct((1, 1), jnp.float32),
        ],
    )(enc_flat, emb_pad, patches, W_dec, b_dec.reshape(1, PATCH_DIM))

    recon6 = pl.pallas_call(
        _unpatchify_body,
        grid=(B,),
        in_specs=[pl.BlockSpec((1, T, PATCH_DIM), lambda i: (i, 0, 0))],
        out_specs=pl.BlockSpec((1, C, GH, P, GW, P),
                               lambda i: (i, 0, 0, 0, 0, 0)),
        out_shape=jax.ShapeDtypeStruct((B, C, GH, P, GW, P), jnp.float32),
    )(dec_flat.reshape(B, T, PATCH_DIM))

    recon = recon6.reshape(B, C, H, W)
    total_vq_loss = sse[0, 0] / (BT * D) * (1.0 + commitment)
    recon_loss = sae[0, 0] / (B * C * H * W)
    overall = total_vq_loss + recon_loss
    embedded_pt = emb_flat.reshape(B, T, D)
    return (overall, total_vq_loss, recon_loss, recon, embedded_pt, idxs)


# speedup vs baseline: 19.1545x; 1.0855x over previous
import jax
import jax.numpy as jnp
from jax.experimental import pallas as pl
from jax.experimental.pallas import tpu as pltpu
from jax.experimental.pallas import tpu_sc as plsc

B, C, H, W = 16, 3, 224, 224
P = 14
K = 8192
D = 64
GH, GW = H // P, W // P
T = GH * GW
PATCH_DIM = C * P * P
BT = B * T

BLK = 256
NBLK = BT // BLK
KBLK = K // NBLK
GATHER_WIN = 128
GATHER_DIM = 128


def _encode_body(in_ref, we_ref, be_ref, cb_ref, enc_ref, p_ref, cbp_ref):
    x = in_ref[0]
    patches = x.transpose(1, 3, 0, 2, 4).reshape(T, PATCH_DIM)
    p_ref[0] = patches
    enc_ref[...] = jnp.dot(patches, we_ref[...],
                           preferred_element_type=jnp.float32) + be_ref[...]
    cb = cb_ref[...]
    cbp_ref[...] = jnp.concatenate([cb, jnp.zeros_like(cb)], axis=1)


def _argmin_body(enc_ref, cbt_ref, rn_ref, cbn_ref, idx_ref):
    d2 = (rn_ref[...]
          - 2.0 * jnp.dot(enc_ref[...], cbt_ref[...],
                          preferred_element_type=jnp.float32)
          + cbn_ref[...])
    m = jnp.min(d2, axis=1, keepdims=True)
    iota = jax.lax.broadcasted_iota(jnp.int32, d2.shape, 1).astype(jnp.float32)
    idx = jnp.min(jnp.where(d2 == m, iota, jnp.float32(K)), axis=1,
                  keepdims=True)
    idx_ref[...] = idx.astype(jnp.int32)


def _decode_loss_body(enc_ref, embp_ref, p_ref, wd_ref, bd_ref,
                      emb_ref, rec_ref, sse_ref, sae_ref):
    emb = embp_ref[:, :D]
    emb_ref[...] = emb
    dec = jnp.dot(emb, wd_ref[...],
                  preferred_element_type=jnp.float32) + bd_ref[...]
    rec_ref[0] = dec.reshape(GH, GW, C, P, P).transpose(2, 0, 3, 1, 4)

    @pl.when(pl.program_id(0) == 0)
    def _():
        sse_ref[...] = jnp.zeros_like(sse_ref)
        sae_ref[...] = jnp.zeros_like(sae_ref)

    diff = enc_ref[...] - emb
    sse_ref[...] += jnp.sum(diff * diff).reshape(1, 1)
    sae_ref[...] += jnp.sum(jnp.abs(dec - p_ref[0])).reshape(1, 1)


def _sc_gather(cb_pad, idx_row):
    mesh = plsc.VectorSubcoreMesh(core_axis_name="core",
                                  subcore_axis_name="subcore")

    @pl.kernel(out_type=jax.ShapeDtypeStruct((BT, GATHER_DIM), jnp.float32),
               mesh=mesh)
    def k(cb_hbm, i_hbm, o_hbm):
        def body(i_vmem, o_vmem):
            pltpu.sync_copy(cb_hbm.at[i_vmem.at[0]], o_vmem)

        pltpu.emit_pipeline(
            body,
            grid=(BT // GATHER_WIN,),
            in_specs=[pl.BlockSpec((1, GATHER_WIN), index_map=lambda i: (0, i))],
            out_specs=[pl.BlockSpec((GATHER_WIN, GATHER_DIM),
                                    index_map=lambda i: (i, 0))],
            core_axis_name=("core", "subcore"),
            dimension_semantics=(pltpu.PARALLEL,),
        )(i_hbm, o_hbm)

    return k(cb_pad, idx_row)


def kernel(inputs, W_enc, b_enc, codebook, W_dec, b_dec, commitment):
    inputs6 = inputs.reshape(B, C, GH, P, GW, P)

    enc_flat, patches3, cb_pad = pl.pallas_call(
        _encode_body,
        grid=(B,),
        in_specs=[
            pl.BlockSpec((1, C, GH, P, GW, P), lambda i: (i, 0, 0, 0, 0, 0)),
            pl.BlockSpec((PATCH_DIM, D), lambda i: (0, 0)),
            pl.BlockSpec((1, D), lambda i: (0, 0)),
            pl.BlockSpec((KBLK, D), lambda i: (i, 0)),
        ],
        out_specs=[
            pl.BlockSpec((BLK, D), lambda i: (i, 0)),
            pl.BlockSpec((1, T, PATCH_DIM), lambda i: (i, 0, 0)),
            pl.BlockSpec((KBLK, GATHER_DIM), lambda i: (i, 0)),
        ],
        out_shape=[
            jax.ShapeDtypeStruct((BT, D), jnp.float32),
            jax.ShapeDtypeStruct((B, T, PATCH_DIM), jnp.float32),
            jax.ShapeDtypeStruct((K, GATHER_DIM), jnp.float32),
        ],
    )(inputs6, W_enc, b_enc.reshape(1, D), codebook)

    rn = jnp.sum(enc_flat * enc_flat, axis=1, keepdims=True)
    cbn = jnp.sum(codebook * codebook, axis=1)[None, :]

    idx_col = pl.pallas_call(
        _argmin_body,
        grid=(NBLK,),
        in_specs=[
            pl.BlockSpec((BLK, D), lambda i: (i, 0)),
            pl.BlockSpec((D, K), lambda i: (0, 0)),
            pl.BlockSpec((BLK, 1), lambda i: (i, 0)),
            pl.BlockSpec((1, K), lambda i: (0, 0)),
        ],
        out_specs=pl.BlockSpec((BLK, 1), lambda i: (i, 0)),
        out_shape=jax.ShapeDtypeStruct((BT, 1), jnp.int32),
    )(enc_flat, codebook.T, rn, cbn)

    idxs = idx_col.reshape(B, T)

    emb_pad = _sc_gather(cb_pad, idx_col.reshape(1, BT))

    emb_flat, recon6, sse, sae = pl.pallas_call(
        _decode_loss_body,
        grid=(B,),
        in_specs=[
            pl.BlockSpec((BLK, D), lambda i: (i, 0)),
            pl.BlockSpec((BLK, GATHER_DIM), lambda i: (i, 0)),
            pl.BlockSpec((1, T, PATCH_DIM), lambda i: (i, 0, 0)),
            pl.BlockSpec((D, PATCH_DIM), lambda i: (0, 0)),
            pl.BlockSpec((1, PATCH_DIM), lambda i: (0, 0)),
        ],
        out_specs=[
            pl.BlockSpec((BLK, D), lambda i: (i, 0)),
            pl.BlockSpec((1, C, GH, P, GW, P), lambda i: (i, 0, 0, 0, 0, 0)),
            pl.BlockSpec((1, 1), lambda i: (0, 0)),
            pl.BlockSpec((1, 1), lambda i: (0, 0)),
        ],
        out_shape=[
            jax.ShapeDtypeStruct((BT, D), jnp.float32),
            jax.ShapeDtypeStruct((B, C, GH, P, GW, P), jnp.float32),
            jax.ShapeDtypeStruct((1, 1), jnp.float32),
            jax.ShapeDtypeStruct((1, 1), jnp.float32),
        ],
    )(enc_flat, emb_pad, patches3, W_dec, b_dec.reshape(1, PATCH_DIM))

    recon = recon6.reshape(B, C, H, W)
    total_vq_loss = sse[0, 0] / (BT * D) * (1.0 + commitment)
    recon_loss = sae[0, 0] / (B * C * H * W)
    overall = total_vq_loss + recon_loss
    embedded_pt = emb_flat.reshape(B, T, D)
    return (overall, total_vq_loss, recon_loss, recon, embedded_pt, idxs)


# speedup vs baseline: 19.3214x; 1.0087x over previous
import jax
import jax.numpy as jnp
from jax.experimental import pallas as pl
from jax.experimental.pallas import tpu as pltpu
from jax.experimental.pallas import tpu_sc as plsc

B, C, H, W = 16, 3, 224, 224
P = 14
K = 8192
D = 64
GH, GW = H // P, W // P
T = GH * GW
PATCH_DIM = C * P * P
BT = B * T

ABLK = 512
BLK = 256
NBLK = BT // BLK
KBLK = K // NBLK
GATHER_WIN = 128
GATHER_DIM = 128


def _encode_body(in_ref, we_ref, be_ref, cb_ref, enc_ref, p_ref, cbp_ref):
    x = in_ref[0]
    patches = x.transpose(1, 3, 0, 2, 4).reshape(T, PATCH_DIM)
    p_ref[0] = patches
    enc_ref[...] = jnp.dot(patches, we_ref[...],
                           preferred_element_type=jnp.float32) + be_ref[...]
    cb = cb_ref[...]
    cbp_ref[...] = jnp.concatenate([cb, jnp.zeros_like(cb)], axis=1)


def _argmin_body(enc_ref, cbt_ref, rn_ref, cbn_ref, idx_ref):
    d2 = (rn_ref[...]
          - 2.0 * jnp.dot(enc_ref[...], cbt_ref[...],
                          preferred_element_type=jnp.float32)
          + cbn_ref[...])
    m = jnp.min(d2, axis=1, keepdims=True)
    iota = jax.lax.broadcasted_iota(jnp.int32, d2.shape, 1).astype(jnp.float32)
    idx = jnp.min(jnp.where(d2 == m, iota, jnp.float32(K)), axis=1,
                  keepdims=True)
    idx_ref[...] = idx.astype(jnp.int32)


def _decode_loss_body(enc_ref, embp_ref, p_ref, wd_ref, bd_ref,
                      emb_ref, rec_ref, sse_ref, sae_ref):
    emb = embp_ref[:, :D]
    emb_ref[...] = emb
    dec = jnp.dot(emb, wd_ref[...],
                  preferred_element_type=jnp.float32) + bd_ref[...]
    rec_ref[0] = dec.reshape(GH, GW, C, P, P).transpose(2, 0, 3, 1, 4)

    @pl.when(pl.program_id(0) == 0)
    def _():
        sse_ref[...] = jnp.zeros_like(sse_ref)
        sae_ref[...] = jnp.zeros_like(sae_ref)

    diff = enc_ref[...] - emb
    sse_ref[...] += jnp.sum(diff * diff).reshape(1, 1)
    sae_ref[...] += jnp.sum(jnp.abs(dec - p_ref[0])).reshape(1, 1)


def _sc_gather(cb_pad, idx_row):
    mesh = plsc.VectorSubcoreMesh(core_axis_name="core",
                                  subcore_axis_name="subcore")

    @pl.kernel(out_type=jax.ShapeDtypeStruct((BT, GATHER_DIM), jnp.float32),
               mesh=mesh)
    def k(cb_hbm, i_hbm, o_hbm):
        def body(i_vmem, o_vmem):
            pltpu.sync_copy(cb_hbm.at[i_vmem.at[0]], o_vmem)

        pltpu.emit_pipeline(
            body,
            grid=(BT // GATHER_WIN,),
            in_specs=[pl.BlockSpec((1, GATHER_WIN), index_map=lambda i: (0, i))],
            out_specs=[pl.BlockSpec((GATHER_WIN, GATHER_DIM),
                                    index_map=lambda i: (i, 0))],
            core_axis_name=("core", "subcore"),
            dimension_semantics=(pltpu.PARALLEL,),
        )(i_hbm, o_hbm)

    return k(cb_pad, idx_row)


def kernel(inputs, W_enc, b_enc, codebook, W_dec, b_dec, commitment):
    inputs6 = inputs.reshape(B, C, GH, P, GW, P)

    enc_flat, patches3, cb_pad = pl.pallas_call(
        _encode_body,
        grid=(B,),
        in_specs=[
            pl.BlockSpec((1, C, GH, P, GW, P), lambda i: (i, 0, 0, 0, 0, 0)),
            pl.BlockSpec((PATCH_DIM, D), lambda i: (0, 0)),
            pl.BlockSpec((1, D), lambda i: (0, 0)),
            pl.BlockSpec((KBLK, D), lambda i: (i, 0)),
        ],
        out_specs=[
            pl.BlockSpec((BLK, D), lambda i: (i, 0)),
            pl.BlockSpec((1, T, PATCH_DIM), lambda i: (i, 0, 0)),
            pl.BlockSpec((KBLK, GATHER_DIM), lambda i: (i, 0)),
        ],
        out_shape=[
            jax.ShapeDtypeStruct((BT, D), jnp.float32),
            jax.ShapeDtypeStruct((B, T, PATCH_DIM), jnp.float32),
            jax.ShapeDtypeStruct((K, GATHER_DIM), jnp.float32),
        ],
    )(inputs6, W_enc, b_enc.reshape(1, D), codebook)

    rn = jnp.sum(enc_flat * enc_flat, axis=1, keepdims=True)
    cbn = jnp.sum(codebook * codebook, axis=1)[None, :]

    idx_col = pl.pallas_call(
        _argmin_body,
        grid=(BT // ABLK,),
        in_specs=[
            pl.BlockSpec((ABLK, D), lambda i: (i, 0)),
            pl.BlockSpec((D, K), lambda i: (0, 0)),
            pl.BlockSpec((ABLK, 1), lambda i: (i, 0)),
            pl.BlockSpec((1, K), lambda i: (0, 0)),
        ],
        out_specs=pl.BlockSpec((ABLK, 1), lambda i: (i, 0)),
        out_shape=jax.ShapeDtypeStruct((BT, 1), jnp.int32),
    )(enc_flat, codebook.T, rn, cbn)

    idxs = idx_col.reshape(B, T)

    emb_pad = _sc_gather(cb_pad, idx_col.reshape(1, BT))

    emb_flat, recon6, sse, sae = pl.pallas_call(
        _decode_loss_body,
        grid=(B,),
        in_specs=[
            pl.BlockSpec((BLK, D), lambda i: (i, 0)),
            pl.BlockSpec((BLK, GATHER_DIM), lambda i: (i, 0)),
            pl.BlockSpec((1, T, PATCH_DIM), lambda i: (i, 0, 0)),
            pl.BlockSpec((D, PATCH_DIM), lambda i: (0, 0)),
            pl.BlockSpec((1, PATCH_DIM), lambda i: (0, 0)),
        ],
        out_specs=[
            pl.BlockSpec((BLK, D), lambda i: (i, 0)),
            pl.BlockSpec((1, C, GH, P, GW, P), lambda i: (i, 0, 0, 0, 0, 0)),
            pl.BlockSpec((1, 1), lambda i: (0, 0)),
            pl.BlockSpec((1, 1), lambda i: (0, 0)),
        ],
        out_shape=[
            jax.ShapeDtypeStruct((BT, D), jnp.float32),
            jax.ShapeDtypeStruct((B, C, GH, P, GW, P), jnp.float32),
            jax.ShapeDtypeStruct((1, 1), jnp.float32),
            jax.ShapeDtypeStruct((1, 1), jnp.float32),
        ],
    )(enc_flat, emb_pad, patches3, W_dec, b_dec.reshape(1, PATCH_DIM))

    recon = recon6.reshape(B, C, H, W)
    total_vq_loss = sse[0, 0] / (BT * D) * (1.0 + commitment)
    recon_loss = sae[0, 0] / (B * C * H * W)
    overall = total_vq_loss + recon_loss
    embedded_pt = emb_flat.reshape(B, T, D)
    return (overall, total_vq_loss, recon_loss, recon, embedded_pt, idxs)


# speedup vs baseline: 19.3782x; 1.0029x over previous
import jax
import jax.numpy as jnp
from jax.experimental import pallas as pl
from jax.experimental.pallas import tpu as pltpu
from jax.experimental.pallas import tpu_sc as plsc

B, C, H, W = 16, 3, 224, 224
P = 14
K = 8192
D = 64
GH, GW = H // P, W // P
T = GH * GW
PATCH_DIM = C * P * P
BT = B * T

ABLK = 1024
BLK = 256
NBLK = BT // BLK
KBLK = K // NBLK
GATHER_WIN = 128
GATHER_DIM = 128


def _encode_body(in_ref, we_ref, be_ref, cb_ref, enc_ref, p_ref, cbp_ref):
    x = in_ref[0]
    patches = x.transpose(1, 3, 0, 2, 4).reshape(T, PATCH_DIM)
    p_ref[0] = patches
    enc_ref[...] = jnp.dot(patches, we_ref[...],
                           preferred_element_type=jnp.float32) + be_ref[...]
    cb = cb_ref[...]
    cbp_ref[...] = jnp.concatenate([cb, jnp.zeros_like(cb)], axis=1)


def _argmin_body(enc_ref, cbt_ref, rn_ref, cbn_ref, idx_ref):
    d2 = (rn_ref[...]
          - 2.0 * jnp.dot(enc_ref[...], cbt_ref[...],
                          preferred_element_type=jnp.float32)
          + cbn_ref[...])
    m = jnp.min(d2, axis=1, keepdims=True)
    iota = jax.lax.broadcasted_iota(jnp.int32, d2.shape, 1).astype(jnp.float32)
    idx = jnp.min(jnp.where(d2 == m, iota, jnp.float32(K)), axis=1,
                  keepdims=True)
    idx_ref[...] = idx.astype(jnp.int32)


def _decode_loss_body(enc_ref, embp_ref, p_ref, wd_ref, bd_ref,
                      emb_ref, rec_ref, sse_ref, sae_ref):
    emb = embp_ref[:, :D]
    emb_ref[...] = emb
    dec = jnp.dot(emb, wd_ref[...],
                  preferred_element_type=jnp.float32) + bd_ref[...]
    rec_ref[0] = dec.reshape(GH, GW, C, P, P).transpose(2, 0, 3, 1, 4)

    @pl.when(pl.program_id(0) == 0)
    def _():
        sse_ref[...] = jnp.zeros_like(sse_ref)
        sae_ref[...] = jnp.zeros_like(sae_ref)

    diff = enc_ref[...] - emb
    sse_ref[...] += jnp.sum(diff * diff).reshape(1, 1)
    sae_ref[...] += jnp.sum(jnp.abs(dec - p_ref[0])).reshape(1, 1)


def _sc_gather(cb_pad, idx_row):
    mesh = plsc.VectorSubcoreMesh(core_axis_name="core",
                                  subcore_axis_name="subcore")

    @pl.kernel(out_type=jax.ShapeDtypeStruct((BT, GATHER_DIM), jnp.float32),
               mesh=mesh)
    def k(cb_hbm, i_hbm, o_hbm):
        def body(i_vmem, o_vmem):
            pltpu.sync_copy(cb_hbm.at[i_vmem.at[0]], o_vmem)

        pltpu.emit_pipeline(
            body,
            grid=(BT // GATHER_WIN,),
            in_specs=[pl.BlockSpec((1, GATHER_WIN), index_map=lambda i: (0, i))],
            out_specs=[pl.BlockSpec((GATHER_WIN, GATHER_DIM),
                                    index_map=lambda i: (i, 0))],
            core_axis_name=("core", "subcore"),
            dimension_semantics=(pltpu.PARALLEL,),
        )(i_hbm, o_hbm)

    return k(cb_pad, idx_row)


def kernel(inputs, W_enc, b_enc, codebook, W_dec, b_dec, commitment):
    inputs6 = inputs.reshape(B, C, GH, P, GW, P)

    enc_flat, patches3, cb_pad = pl.pallas_call(
        _encode_body,
        grid=(B,),
        in_specs=[
            pl.BlockSpec((1, C, GH, P, GW, P), lambda i: (i, 0, 0, 0, 0, 0)),
            pl.BlockSpec((PATCH_DIM, D), lambda i: (0, 0)),
            pl.BlockSpec((1, D), lambda i: (0, 0)),
            pl.BlockSpec((KBLK, D), lambda i: (i, 0)),
        ],
        out_specs=[
            pl.BlockSpec((BLK, D), lambda i: (i, 0)),
            pl.BlockSpec((1, T, PATCH_DIM), lambda i: (i, 0, 0)),
            pl.BlockSpec((KBLK, GATHER_DIM), lambda i: (i, 0)),
        ],
        out_shape=[
            jax.ShapeDtypeStruct((BT, D), jnp.float32),
            jax.ShapeDtypeStruct((B, T, PATCH_DIM), jnp.float32),
            jax.ShapeDtypeStruct((K, GATHER_DIM), jnp.float32),
        ],
    )(inputs6, W_enc, b_enc.reshape(1, D), codebook)

    rn = jnp.sum(enc_flat * enc_flat, axis=1, keepdims=True)
    cbn = jnp.sum(codebook * codebook, axis=1)[None, :]

    idx_col = pl.pallas_call(
        _argmin_body,
        grid=(BT // ABLK,),
        in_specs=[
            pl.BlockSpec((ABLK, D), lambda i: (i, 0)),
            pl.BlockSpec((D, K), lambda i: (0, 0)),
            pl.BlockSpec((ABLK, 1), lambda i: (i, 0)),
            pl.BlockSpec((1, K), lambda i: (0, 0)),
        ],
        out_specs=pl.BlockSpec((ABLK, 1), lambda i: (i, 0)),
        out_shape=jax.ShapeDtypeStruct((BT, 1), jnp.int32),
    )(enc_flat, codebook.T, rn, cbn)

    idxs = idx_col.reshape(B, T)

    emb_pad = _sc_gather(cb_pad, idx_col.reshape(1, BT))

    emb_flat, recon6, sse, sae = pl.pallas_call(
        _decode_loss_body,
        grid=(B,),
        in_specs=[
            pl.BlockSpec((BLK, D), lambda i: (i, 0)),
            pl.BlockSpec((BLK, GATHER_DIM), lambda i: (i, 0)),
            pl.BlockSpec((1, T, PATCH_DIM), lambda i: (i, 0, 0)),
            pl.BlockSpec((D, PATCH_DIM), lambda i: (0, 0)),
            pl.BlockSpec((1, PATCH_DIM), lambda i: (0, 0)),
        ],
        out_specs=[
            pl.BlockSpec((BLK, D), lambda i: (i, 0)),
            pl.BlockSpec((1, C, GH, P, GW, P), lambda i: (i, 0, 0, 0, 0, 0)),
            pl.BlockSpec((1, 1), lambda i: (0, 0)),
            pl.BlockSpec((1, 1), lambda i: (0, 0)),
        ],
        out_shape=[
            jax.ShapeDtypeStruct((BT, D), jnp.float32),
            jax.ShapeDtypeStruct((B, C, GH, P, GW, P), jnp.float32),
            jax.ShapeDtypeStruct((1, 1), jnp.float32),
            jax.ShapeDtypeStruct((1, 1), jnp.float32),
        ],
    )(enc_flat, emb_pad, patches3, W_dec, b_dec.reshape(1, PATCH_DIM))

    recon = recon6.reshape(B, C, H, W)
    total_vq_loss = sse[0, 0] / (BT * D) * (1.0 + commitment)
    recon_loss = sae[0, 0] / (B * C * H * W)
    overall = total_vq_loss + recon_loss
    embedded_pt = emb_flat.reshape(B, T, D)
    return (overall, total_vq_loss, recon_loss, recon, embedded_pt, idxs)
